# baseline (speedup 1.0000x reference)
import functools
import math

import jax
import jax.numpy as jnp
from jax import lax
from jax.experimental import pallas as pl
from jax.experimental.pallas import tpu as pltpu
from jax.experimental.pallas import tpu_sc as plsc

NV = 4096
E = 65536
BATCH = 8
KB = 5
HID = 16
NT = 16
NC = 2
BH = BATCH // NC
EPT = E // NT
NR = NV // NT
NCHUNK = EPT // 128

_f32 = jnp.float32
_i32 = jnp.int32


def _sc_body(x_hbm, ei_hbm, w1_hbm, b1_hbm, w2_hbm, out_hbm,
             x4, src1, dst2, w1v, b1v, w2v, sem, dinv_loc, dbuf, msg,
             gloc, sbuf, zfin, s_sh, z_sh, g_sh, dinv_sh):
    c = lax.axis_index("c")
    t = lax.axis_index("s")
    r0 = t * NR
    e0 = t * EPT

    iot = lax.iota(_i32, 16)
    rpat = iot >> 2
    cpat = iot & 3
    zeros16i = jnp.zeros((16,), _i32)
    zeros16 = jnp.zeros((16,), _f32)
    ones16 = jnp.ones((16,), _f32)

    pltpu.sync_copy(x_hbm.at[pl.ds(c * BH, BH)], x4)
    pltpu.sync_copy(ei_hbm.at[0, pl.ds(e0, EPT)], src1)
    copies = [pltpu.async_copy(ei_hbm.at[1, pl.ds(e0 + j * 128, 128)],
                               dst2.at[j], sem)
              for j in range(NCHUNK)]
    pltpu.sync_copy(w1_hbm, w1v)
    pltpu.sync_copy(b1_hbm, b1v)
    pltpu.sync_copy(w2_hbm, w2v)
    for cp in copies:
        cp.wait()

    def _zero_sbuf(i, _):
        plsc.store_scatter(sbuf, [rpat + i * 4, cpat], zeros16)
        return _
    lax.fori_loop(0, NR * 4 // 16, _zero_sbuf, None)
    pltpu.sync_copy(sbuf, s_sh.at[pl.ds(r0, NR)])
    pltpu.sync_copy(sbuf, z_sh.at[pl.ds(r0, NR)])
    pltpu.sync_copy(sbuf, g_sh.at[pl.ds(r0, NR)])

    def _fill_ones(i, _):
        plsc.store_scatter(msg, [rpat + i * 4, cpat], ones16)
        return _
    lax.fori_loop(0, EPT * 4 // 16, _fill_ones, None)
    plsc.subcore_barrier()

    for j in range(NCHUNK):
        pltpu.sync_copy(msg.at[pl.ds(j * 128, 128)], g_sh.at[dst2.at[j]],
                        add=True)
    plsc.subcore_barrier()

    pltpu.sync_copy(g_sh.at[pl.ds(r0, NR)], sbuf)

    def _dinv_step(i, _):
        d = plsc.load_gather(sbuf, [i * 16 + iot, zeros16i])
        d = d + 1.0
        yi = 0x5F3759DF - (plsc.bitcast(d, _i32) >> 1)
        y = plsc.bitcast(yi, _f32)
        for _it in range(4):
            y = y * (1.5 - 0.5 * d * y * y)
        dbuf[pl.ds(i * 16, 16)] = y
        return _
    lax.fori_loop(0, NR // 16, _dinv_step, None)
    pltpu.sync_copy(dbuf, dinv_sh.at[pl.ds(r0, NR)])
    plsc.subcore_barrier()
    pltpu.sync_copy(dinv_sh, dinv_loc)

    def _compute_msgs(val_gather):
        def body(i, _):
            eidx = i * 4 + rpat
            srcv = plsc.load_gather(src1, [eidx])
            dstv = plsc.load_gather(dst2, [eidx >> 7, eidx & 127])
            nrm = (plsc.load_gather(dinv_loc, [srcv]) *
                   plsc.load_gather(dinv_loc, [dstv]))
            vals = val_gather(srcv)
            plsc.store_scatter(msg, [eidx, cpat], vals * nrm)
            return _
        lax.fori_loop(0, EPT // 4, body, None)

    _compute_msgs(lambda srcv: plsc.load_gather(x4, [cpat, srcv]))
    for j in range(NCHUNK):
        pltpu.sync_copy(msg.at[pl.ds(j * 128, 128)], s_sh.at[dst2.at[j]],
                        add=True)
    plsc.subcore_barrier()

    pltpu.sync_copy(s_sh.at[pl.ds(r0, NR)], sbuf)
    a2s = [2.0 * plsc.load_gather(w1v, [zeros16i, zeros16i + j])
           for j in range(HID)]
    b2s = [2.0 * plsc.load_gather(b1v, [zeros16i + j]) for j in range(HID)]
    cs = [plsc.load_gather(w2v, [zeros16i + j, zeros16i]) for j in range(HID)]
    s0 = cs[0]
    for j in range(1, HID):
        s0 = s0 + cs[j]

    def _f_step(i, _):
        rr = rpat + i * 4
        sv = plsc.load_gather(sbuf, [rr, cpat])
        xv = plsc.load_gather(x4, [cpat, r0 + rr])
        di = plsc.load_gather(dbuf, [rr])
        s_full = sv + di * di * xv
        acc = jnp.zeros((16,), _f32)
        for j in range(HID):
            ej = jnp.exp(s_full * a2s[j] + b2s[j])
            acc = acc + cs[j] / (ej + 1.0)
        plsc.store_scatter(zfin, [rr, cpat], s0 - 2.0 * acc)
        return _
    lax.fori_loop(0, NR * 4 // 16, _f_step, None)
    pltpu.sync_copy(zfin, g_sh.at[pl.ds(r0, NR)])
    plsc.subcore_barrier()
    pltpu.sync_copy(g_sh, gloc)

    _compute_msgs(lambda srcv: plsc.load_gather(gloc, [srcv, cpat]))
    for j in range(NCHUNK):
        pltpu.sync_copy(msg.at[pl.ds(j * 128, 128)], z_sh.at[dst2.at[j]],
                        add=True)
    plsc.subcore_barrier()

    pltpu.sync_copy(z_sh.at[pl.ds(r0, NR)], sbuf)

    def _fin_step(i, _):
        rr = rpat + i * 4
        zv = plsc.load_gather(sbuf, [rr, cpat])
        gv = plsc.load_gather(gloc, [r0 + rr, cpat])
        di = plsc.load_gather(dbuf, [rr])
        plsc.store_scatter(zfin, [rr, cpat], zv + di * di * gv)
        return _
    lax.fori_loop(0, NR * 4 // 16, _fin_step, None)
    pltpu.sync_copy(zfin, out_hbm.at[c, pl.ds(r0, NR)])


def _graph_sc(x, ei, W1, b1, W2):
    mesh = plsc.VectorSubcoreMesh(core_axis_name="c", subcore_axis_name="s")
    f = pl.kernel(
        _sc_body,
        out_type=jax.ShapeDtypeStruct((NC, NV, BH), _f32),
        mesh=mesh,
        compiler_params=pltpu.CompilerParams(
            needs_layout_passes=False, use_tc_tiling_on_sc=False),
        scratch_types=[
            pltpu.VMEM((BH, NV), _f32),
            pltpu.VMEM((EPT,), _i32),
            pltpu.VMEM((NCHUNK, 128), _i32),
            pltpu.VMEM((1, HID), _f32),
            pltpu.VMEM((HID,), _f32),
            pltpu.VMEM((HID, 1), _f32),
            pltpu.SemaphoreType.DMA,
            pltpu.VMEM((NV,), _f32),
            pltpu.VMEM((NR,), _f32),
            pltpu.VMEM((EPT, BH), _f32),
            pltpu.VMEM((NV, BH), _f32),
            pltpu.VMEM((NR, BH), _f32),
            pltpu.VMEM((NR, BH), _f32),
            pltpu.VMEM_SHARED((NV, BH), _f32),
            pltpu.VMEM_SHARED((NV, BH), _f32),
            pltpu.VMEM_SHARED((NV, BH), _f32),
            pltpu.VMEM_SHARED((NV,), _f32),
        ],
    )
    return f(x, ei, W1, b1, W2)


TILE_U = 1024
OMEGA = 2.0 * math.pi


def _dxdt_body(t_ref, aw_ref, L_ref, x_ref, out_ref, al_ref):
    k = pl.program_id(1)
    tv = t_ref[...]
    freq = ((k + 1) // 2).astype(_f32)
    ang = OMEGA * freq * tv
    embk = jnp.where(k == 0, jnp.ones_like(tv),
                     jnp.where(k % 2 == 1, jnp.cos(ang), jnp.sin(ang)))
    mm = lax.dot_general(L_ref[0], x_ref[...],
                         (((1,), (1,)), ((), ())),
                         preferred_element_type=_f32)
    contrib = mm * embk

    @pl.when(k == 0)
    def _():
        out_ref[...] = contrib

    @pl.when(k > 0)
    def _():
        out_ref[...] += contrib

    @pl.when(k == KB - 1)
    def _():
        acc = jnp.zeros_like(tv) + aw_ref[0]
        acc += aw_ref[1] * jnp.cos(OMEGA * tv)
        acc += aw_ref[2] * jnp.sin(OMEGA * tv)
        acc += aw_ref[3] * jnp.cos(2.0 * OMEGA * tv)
        acc += aw_ref[4] * jnp.sin(2.0 * OMEGA * tv)
        al_ref[...] = jax.nn.sigmoid(acc)


def _dxdt(t_years, alpha_w, L_basis, x):
    grid = (NV // TILE_U, KB)
    return pl.pallas_call(
        _dxdt_body,
        grid=grid,
        in_specs=[
            pl.BlockSpec((1, BATCH), lambda u, k: (0, 0)),
            pl.BlockSpec(memory_space=pltpu.SMEM),
            pl.BlockSpec((1, TILE_U, NV), lambda u, k: (k, u, 0)),
            pl.BlockSpec((BATCH, NV), lambda u, k: (0, 0)),
        ],
        out_specs=[
            pl.BlockSpec((TILE_U, BATCH), lambda u, k: (u, 0)),
            pl.BlockSpec((1, BATCH), lambda u, k: (0, 0)),
        ],
        out_shape=[
            jax.ShapeDtypeStruct((NV, BATCH), _f32),
            jax.ShapeDtypeStruct((1, BATCH), _f32),
        ],
    )(t_years.reshape(1, BATCH), alpha_w, L_basis, x)


TILE_E = 1024


def _epi_body(b2_ref, al_ref, dx_ref, g_ref, out_ref):
    alpha = al_ref[...]
    g = jnp.concatenate([g_ref[0], g_ref[1]], axis=1)
    y = dx_ref[...] + alpha * (g + b2_ref[0])
    out_ref[...] = y.T


def _epilogue(b2, alpha, dxT, ag):
    grid = (NV // TILE_E,)
    return pl.pallas_call(
        _epi_body,
        grid=grid,
        in_specs=[
            pl.BlockSpec(memory_space=pltpu.SMEM),
            pl.BlockSpec((1, BATCH), lambda u: (0, 0)),
            pl.BlockSpec((TILE_E, BATCH), lambda u: (u, 0)),
            pl.BlockSpec((NC, TILE_E, BH), lambda u: (0, u, 0)),
        ],
        out_specs=pl.BlockSpec((BATCH, TILE_E), lambda u: (0, u)),
        out_shape=jax.ShapeDtypeStruct((BATCH, NV), _f32),
    )(b2, alpha, dxT, ag)


@jax.jit
def kernel(x, t_years, edge_index, L_basis, alpha_w, W1, b1, W2, b2):
    ag = _graph_sc(x, edge_index, W1, b1, W2)
    dxT, alpha = _dxdt(t_years, alpha_w, L_basis, x)

    return _epilogue(b2, alpha, dxT, ag)


# speedup vs baseline: 1.0514x; 1.0514x over previous
import functools
import math

import jax
import jax.numpy as jnp
from jax import lax
from jax.experimental import pallas as pl
from jax.experimental.pallas import tpu as pltpu
from jax.experimental.pallas import tpu_sc as plsc

NV = 4096
E = 65536
BATCH = 8
KB = 5
HID = 16
NT = 16
NC = 2
BH = BATCH // NC
EPT = E // NT
NR = NV // NT
NCHUNK = EPT // 128

_f32 = jnp.float32
_i32 = jnp.int32


def _sc_body(x_hbm, ei_hbm, w1_hbm, b1_hbm, w2_hbm, out_hbm,
             x4, src1, dst2, w1v, b1v, w2v, sem, dinv_loc, dbuf, msg,
             gloc, sbuf, zfin, zout, s_sh, z_sh, g_sh, dinv_sh):
    c = lax.axis_index("c")
    t = lax.axis_index("s")
    r0 = t * NR
    e0 = t * EPT

    iot = lax.iota(_i32, 16)
    rpat = iot >> 2
    cpat = iot & 3
    zeros16i = jnp.zeros((16,), _i32)
    zeros16 = jnp.zeros((16,), _f32)
    ones16 = jnp.ones((16,), _f32)

    pltpu.sync_copy(x_hbm.at[pl.ds(c * BH, BH)], x4)
    pltpu.sync_copy(ei_hbm.at[0, pl.ds(e0, EPT)], src1)
    copies = [pltpu.async_copy(ei_hbm.at[1, pl.ds(e0 + j * 128, 128)],
                               dst2.at[j], sem)
              for j in range(NCHUNK)]
    pltpu.sync_copy(w1_hbm, w1v)
    pltpu.sync_copy(b1_hbm, b1v)
    pltpu.sync_copy(w2_hbm, w2v)
    for cp in copies:
        cp.wait()

    def _zero_sbuf(i, _):
        plsc.store_scatter(sbuf, [rpat + i * 4, cpat], zeros16)
        return _
    lax.fori_loop(0, NR * 4 // 16, _zero_sbuf, None)
    pltpu.sync_copy(sbuf, s_sh.at[pl.ds(r0, NR)])
    pltpu.sync_copy(sbuf, z_sh.at[pl.ds(r0, NR)])
    pltpu.sync_copy(sbuf, g_sh.at[pl.ds(r0, NR)])

    def _fill_ones(i, _):
        plsc.store_scatter(msg, [rpat + i * 4, cpat], ones16)
        return _
    lax.fori_loop(0, EPT * 4 // 16, _fill_ones, None)
    plsc.subcore_barrier()

    for j in range(NCHUNK):
        pltpu.sync_copy(msg.at[pl.ds(j * 128, 128)], g_sh.at[dst2.at[j]],
                        add=True)
    plsc.subcore_barrier()

    pltpu.sync_copy(g_sh.at[pl.ds(r0, NR)], sbuf)

    def _dinv_step(i, _):
        d = plsc.load_gather(sbuf, [i * 16 + iot, zeros16i])
        d = d + 1.0
        yi = 0x5F3759DF - (plsc.bitcast(d, _i32) >> 1)
        y = plsc.bitcast(yi, _f32)
        for _it in range(4):
            y = y * (1.5 - 0.5 * d * y * y)
        dbuf[pl.ds(i * 16, 16)] = y
        return _
    lax.fori_loop(0, NR // 16, _dinv_step, None)
    pltpu.sync_copy(dbuf, dinv_sh.at[pl.ds(r0, NR)])
    plsc.subcore_barrier()
    pltpu.sync_copy(dinv_sh, dinv_loc)

    def _compute_msgs(val_gather):
        def body(i, _):
            eidx = i * 4 + rpat
            srcv = plsc.load_gather(src1, [eidx])
            dstv = plsc.load_gather(dst2, [eidx >> 7, eidx & 127])
            nrm = (plsc.load_gather(dinv_loc, [srcv]) *
                   plsc.load_gather(dinv_loc, [dstv]))
            vals = val_gather(srcv)
            plsc.store_scatter(msg, [eidx, cpat], vals * nrm)
            return _
        lax.fori_loop(0, EPT // 4, body, None)

    _compute_msgs(lambda srcv: plsc.load_gather(x4, [cpat, srcv]))
    for j in range(NCHUNK):
        pltpu.sync_copy(msg.at[pl.ds(j * 128, 128)], s_sh.at[dst2.at[j]],
                        add=True)
    plsc.subcore_barrier()

    pltpu.sync_copy(s_sh.at[pl.ds(r0, NR)], sbuf)
    a2s = [2.0 * plsc.load_gather(w1v, [zeros16i, zeros16i + j])
           for j in range(HID)]
    b2s = [2.0 * plsc.load_gather(b1v, [zeros16i + j]) for j in range(HID)]
    cs = [plsc.load_gather(w2v, [zeros16i + j, zeros16i]) for j in range(HID)]
    s0 = cs[0]
    for j in range(1, HID):
        s0 = s0 + cs[j]

    def _f_step(i, _):
        rr = rpat + i * 4
        sv = plsc.load_gather(sbuf, [rr, cpat])
        xv = plsc.load_gather(x4, [cpat, r0 + rr])
        di = plsc.load_gather(dbuf, [rr])
        s_full = sv + di * di * xv
        acc = jnp.zeros((16,), _f32)
        for j in range(HID):
            ej = jnp.exp(s_full * a2s[j] + b2s[j])
            acc = acc + cs[j] / (ej + 1.0)
        plsc.store_scatter(zfin, [rr, cpat], s0 - 2.0 * acc)
        return _
    lax.fori_loop(0, NR * 4 // 16, _f_step, None)
    pltpu.sync_copy(zfin, g_sh.at[pl.ds(r0, NR)])
    plsc.subcore_barrier()
    pltpu.sync_copy(g_sh, gloc)

    _compute_msgs(lambda srcv: plsc.load_gather(gloc, [srcv, cpat]))
    for j in range(NCHUNK):
        pltpu.sync_copy(msg.at[pl.ds(j * 128, 128)], z_sh.at[dst2.at[j]],
                        add=True)
    plsc.subcore_barrier()

    pltpu.sync_copy(z_sh.at[pl.ds(r0, NR)], sbuf)

    def _fin_step(i, _):
        rr = rpat + i * 4
        zv = plsc.load_gather(sbuf, [rr, cpat])
        gv = plsc.load_gather(gloc, [r0 + rr, cpat])
        di = plsc.load_gather(dbuf, [rr])
        plsc.store_scatter(zout, [cpat, rr], zv + di * di * gv)
        return _
    lax.fori_loop(0, NR * 4 // 16, _fin_step, None)
    pltpu.sync_copy(zout, out_hbm.at[c, :, pl.ds(r0, NR)])


def _graph_sc(x, ei, W1, b1, W2):
    mesh = plsc.VectorSubcoreMesh(core_axis_name="c", subcore_axis_name="s")
    f = pl.kernel(
        _sc_body,
        out_type=jax.ShapeDtypeStruct((NC, BH, NV), _f32),
        mesh=mesh,
        compiler_params=pltpu.CompilerParams(
            needs_layout_passes=False, use_tc_tiling_on_sc=False),
        scratch_types=[
            pltpu.VMEM((BH, NV), _f32),
            pltpu.VMEM((EPT,), _i32),
            pltpu.VMEM((NCHUNK, 128), _i32),
            pltpu.VMEM((1, HID), _f32),
            pltpu.VMEM((HID,), _f32),
            pltpu.VMEM((HID, 1), _f32),
            pltpu.SemaphoreType.DMA,
            pltpu.VMEM((NV,), _f32),
            pltpu.VMEM((NR,), _f32),
            pltpu.VMEM((EPT, BH), _f32),
            pltpu.VMEM((NV, BH), _f32),
            pltpu.VMEM((NR, BH), _f32),
            pltpu.VMEM((NR, BH), _f32),
            pltpu.VMEM((BH, NR), _f32),
            pltpu.VMEM_SHARED((NV, BH), _f32),
            pltpu.VMEM_SHARED((NV, BH), _f32),
            pltpu.VMEM_SHARED((NV, BH), _f32),
            pltpu.VMEM_SHARED((NV,), _f32),
        ],
    )
    return f(x, ei, W1, b1, W2)


TILE_U = 1024
OMEGA = 2.0 * math.pi


def _dxdt_body(t_ref, aw_ref, L_ref, x_ref, out_ref, al_ref):
    k = pl.program_id(1)
    tv = t_ref[...]
    freq = ((k + 1) // 2).astype(_f32)
    ang = OMEGA * freq * tv
    embk = jnp.where(k == 0, jnp.ones_like(tv),
                     jnp.where(k % 2 == 1, jnp.cos(ang), jnp.sin(ang)))
    mm = lax.dot_general(L_ref[0], x_ref[...],
                         (((1,), (1,)), ((), ())),
                         preferred_element_type=_f32)
    contrib = mm * embk

    @pl.when(k == 0)
    def _():
        out_ref[...] = contrib

    @pl.when(k > 0)
    def _():
        out_ref[...] += contrib

    @pl.when(k == KB - 1)
    def _():
        acc = jnp.zeros_like(tv) + aw_ref[0]
        acc += aw_ref[1] * jnp.cos(OMEGA * tv)
        acc += aw_ref[2] * jnp.sin(OMEGA * tv)
        acc += aw_ref[3] * jnp.cos(2.0 * OMEGA * tv)
        acc += aw_ref[4] * jnp.sin(2.0 * OMEGA * tv)
        al_ref[...] = jax.nn.sigmoid(acc)


def _dxdt(t_years, alpha_w, L_basis, x):
    grid = (NV // TILE_U, KB)
    return pl.pallas_call(
        _dxdt_body,
        grid=grid,
        in_specs=[
            pl.BlockSpec((1, BATCH), lambda u, k: (0, 0)),
            pl.BlockSpec(memory_space=pltpu.SMEM),
            pl.BlockSpec((1, TILE_U, NV), lambda u, k: (k, u, 0)),
            pl.BlockSpec((BATCH, NV), lambda u, k: (0, 0)),
        ],
        out_specs=[
            pl.BlockSpec((TILE_U, BATCH), lambda u, k: (u, 0)),
            pl.BlockSpec((1, BATCH), lambda u, k: (0, 0)),
        ],
        out_shape=[
            jax.ShapeDtypeStruct((NV, BATCH), _f32),
            jax.ShapeDtypeStruct((1, BATCH), _f32),
        ],
    )(t_years.reshape(1, BATCH), alpha_w, L_basis, x)


TILE_E = 1024


def _epi_body(b2_ref, al_ref, dx_ref, g_ref, out_ref):
    alpha = al_ref[...]
    g = jnp.concatenate([g_ref[0], g_ref[1]], axis=0)
    y = dx_ref[...] + alpha * (g.T + b2_ref[0])
    out_ref[...] = y.T


def _epilogue(b2, alpha, dxT, ag):
    grid = (NV // TILE_E,)
    return pl.pallas_call(
        _epi_body,
        grid=grid,
        in_specs=[
            pl.BlockSpec(memory_space=pltpu.SMEM),
            pl.BlockSpec((1, BATCH), lambda u: (0, 0)),
            pl.BlockSpec((TILE_E, BATCH), lambda u: (u, 0)),
            pl.BlockSpec((NC, BH, TILE_E), lambda u: (0, 0, u)),
        ],
        out_specs=pl.BlockSpec((BATCH, TILE_E), lambda u: (0, u)),
        out_shape=jax.ShapeDtypeStruct((BATCH, NV), _f32),
    )(b2, alpha, dxT, ag)


@jax.jit
def kernel(x, t_years, edge_index, L_basis, alpha_w, W1, b1, W2, b2):
    ag = _graph_sc(x, edge_index, W1, b1, W2)
    dxT, alpha = _dxdt(t_years, alpha_w, L_basis, x)

    return _epilogue(b2, alpha, dxT, ag)
